# Initial kernel scaffold; baseline (speedup 1.0000x reference)
import functools

import jax
import jax.numpy as jnp
from jax import lax
from jax.experimental import pallas as pl
from jax.experimental.pallas import tpu as pltpu
from jax.experimental.pallas import tpu_sc as plsc

_VOCAB = 262144
_HIDDEN = 1536
_NUM_TOKENS = 4 * 2048

_NUM_CORES = 2
_NUM_SUBCORES = 16
_NW = _NUM_CORES * _NUM_SUBCORES
_B_PER_W = _NUM_TOKENS // _NW
_CHUNK = 32
_NCHUNK = _B_PER_W // _CHUNK
_NBUF = 2


def _make_gather():
  mesh = plsc.VectorSubcoreMesh(core_axis_name="c", subcore_axis_name="s")

  @functools.partial(
      pl.kernel,
      mesh=mesh,
      out_type=jax.ShapeDtypeStruct((_NUM_TOKENS, _HIDDEN), jnp.float32),
      scratch_types=[
          pltpu.VMEM((_NCHUNK, _CHUNK), jnp.int32),
          pltpu.VMEM((_NBUF, _CHUNK, _HIDDEN), jnp.float32),
          pltpu.SemaphoreType.DMA,
          pltpu.SemaphoreType.DMA,
          pltpu.SemaphoreType.DMA,
      ],
  )
  def gather_kernel(idx_hbm, table_hbm, out_hbm, idx_v, rows_v, isem, gsem,
                    ssem):
    wid = lax.axis_index("s") * _NUM_CORES + lax.axis_index("c")
    base = wid * _B_PER_W
    pltpu.async_copy(idx_hbm.at[wid], idx_v, isem).wait()

    gathers = []
    scatters = []
    gathers.append(
        pltpu.async_copy(table_hbm.at[idx_v.at[0]], rows_v.at[0], gsem))
    for c in range(_NCHUNK):
      gathers[c].wait()
      if c + 1 < _NCHUNK:
        if c + 1 >= _NBUF:
          scatters[c + 1 - _NBUF].wait()
        gathers.append(
            pltpu.async_copy(table_hbm.at[idx_v.at[c + 1]],
                             rows_v.at[(c + 1) % _NBUF], gsem))
      scatters.append(
          pltpu.async_copy(rows_v.at[c % _NBUF],
                           out_hbm.at[pl.ds(base + c * _CHUNK, _CHUNK)],
                           ssem))
    for c in range(_NCHUNK - _NBUF + 1, _NCHUNK):
      scatters[c].wait()

  return gather_kernel


_gather = _make_gather()


def kernel(token_ids, table):
  ids = token_ids.astype(jnp.int32).reshape(_NW, _NCHUNK, _CHUNK)
  out = _gather(ids, table)
  return out.reshape(token_ids.shape[0], token_ids.shape[1], _HIDDEN)


# speedup vs baseline: 1.5428x; 1.5428x over previous
import functools

import jax
import jax.numpy as jnp
from jax import lax
from jax.experimental import pallas as pl
from jax.experimental.pallas import tpu as pltpu
from jax.experimental.pallas import tpu_sc as plsc

_VOCAB = 262144
_HIDDEN = 1536
_NUM_TOKENS = 4 * 2048

_NUM_CORES = 2
_NUM_SUBCORES = 16
_NW = _NUM_CORES * _NUM_SUBCORES
_B_PER_W = _NUM_TOKENS // _NW
_CHUNK = 32
_NCHUNK = _B_PER_W // _CHUNK
_NBUF = 2


def _make_gather():
  mesh = plsc.VectorSubcoreMesh(core_axis_name="c", subcore_axis_name="s")

  @functools.partial(
      pl.kernel,
      mesh=mesh,
      out_type=jax.ShapeDtypeStruct((_NUM_TOKENS, _HIDDEN), jnp.float32),
      scratch_types=[
          pltpu.VMEM((_NCHUNK, _CHUNK), jnp.int32),
          pltpu.VMEM((_NBUF, _CHUNK, _HIDDEN), jnp.float32),
          pltpu.SemaphoreType.DMA,
          pltpu.SemaphoreType.DMA,
          pltpu.SemaphoreType.DMA,
      ],
  )
  def gather_kernel(idx_hbm, table_hbm, out_hbm, idx_v, rows_v, isem, gsem,
                    ssem):
    wid = lax.axis_index("s") * _NUM_CORES + lax.axis_index("c")
    base = wid * _B_PER_W
    pltpu.async_copy(idx_hbm.at[wid], idx_v, isem).wait()

    gathers = []
    scatters = []
    gathers.append(
        pltpu.async_copy(table_hbm.at[idx_v.at[0]], rows_v.at[0], gsem))
    for c in range(_NCHUNK):
      gathers[c].wait()
      if c + 1 < _NCHUNK:
        if c + 1 >= _NBUF:
          scatters[c + 1 - _NBUF].wait()
        gathers.append(
            pltpu.async_copy(table_hbm.at[idx_v.at[c + 1]],
                             rows_v.at[(c + 1) % _NBUF], gsem))
      scatters.append(
          pltpu.async_copy(rows_v.at[c % _NBUF],
                           out_hbm.at[pl.ds(base + c * _CHUNK, _CHUNK)],
                           ssem))
    for c in range(_NCHUNK - _NBUF, _NCHUNK):
      scatters[c].wait()

  return gather_kernel


_gather = _make_gather()


def kernel(token_ids, table):
  ids = token_ids.astype(jnp.int32).reshape(_NW, _NCHUNK, _CHUNK)
  out = _gather(ids, table)
  return out.reshape(token_ids.shape[0], token_ids.shape[1], _HIDDEN)


# speedup vs baseline: 1.5848x; 1.0272x over previous
import functools

import jax
import jax.numpy as jnp
from jax import lax
from jax.experimental import pallas as pl
from jax.experimental.pallas import tpu as pltpu
from jax.experimental.pallas import tpu_sc as plsc

_VOCAB = 262144
_HIDDEN = 1536
_NUM_TOKENS = 4 * 2048

_NUM_CORES = 2
_NUM_SUBCORES = 16
_NW = _NUM_CORES * _NUM_SUBCORES
_B_PER_W = _NUM_TOKENS // _NW
_CHUNK = 16
_NCHUNK = _B_PER_W // _CHUNK
_NBUF = 4
_LA = 3


def _make_gather():
  mesh = plsc.VectorSubcoreMesh(core_axis_name="c", subcore_axis_name="s")

  @functools.partial(
      pl.kernel,
      mesh=mesh,
      out_type=jax.ShapeDtypeStruct((_NUM_TOKENS, _HIDDEN), jnp.float32),
      scratch_types=[
          pltpu.VMEM((_NCHUNK, _CHUNK), jnp.int32),
          pltpu.VMEM((_NBUF, _CHUNK, _HIDDEN), jnp.float32),
          pltpu.SemaphoreType.DMA,
          pltpu.SemaphoreType.DMA,
          pltpu.SemaphoreType.DMA,
      ],
  )
  def gather_kernel(idx_hbm, table_hbm, out_hbm, idx_v, rows_v, isem, gsem,
                    ssem):
    wid = lax.axis_index("s") * _NUM_CORES + lax.axis_index("c")
    base = wid * _B_PER_W
    pltpu.async_copy(idx_hbm.at[wid], idx_v, isem).wait()

    gathers = []
    scatters = []

    def start_gather(n):
      gathers.append(
          pltpu.async_copy(table_hbm.at[idx_v.at[n]],
                           rows_v.at[n % _NBUF], gsem))

    for n in range(_LA):
      start_gather(n)
    for c in range(_NCHUNK):
      gathers[c].wait()
      scatters.append(
          pltpu.async_copy(rows_v.at[c % _NBUF],
                           out_hbm.at[pl.ds(base + c * _CHUNK, _CHUNK)],
                           ssem))
      n = c + _LA
      if n < _NCHUNK:
        if n >= _NBUF:
          scatters[n - _NBUF].wait()
        start_gather(n)
    for c in range(_NCHUNK - _NBUF, _NCHUNK):
      scatters[c].wait()

  return gather_kernel


_gather = _make_gather()


def kernel(token_ids, table):
  ids = token_ids.astype(jnp.int32).reshape(_NW, _NCHUNK, _CHUNK)
  out = _gather(ids, table)
  return out.reshape(token_ids.shape[0], token_ids.shape[1], _HIDDEN)


# speedup vs baseline: 1.5936x; 1.0056x over previous
import functools

import jax
import jax.numpy as jnp
from jax import lax
from jax.experimental import pallas as pl
from jax.experimental.pallas import tpu as pltpu
from jax.experimental.pallas import tpu_sc as plsc

_VOCAB = 262144
_HIDDEN = 1536
_NUM_TOKENS = 4 * 2048

_NUM_CORES = 2
_NUM_SUBCORES = 16
_NW = _NUM_CORES * _NUM_SUBCORES
_B_PER_W = _NUM_TOKENS // _NW
_CHUNK = 16
_NCHUNK = _B_PER_W // _CHUNK
_NBUF = 5
_LA = 2


def _make_gather():
  mesh = plsc.VectorSubcoreMesh(core_axis_name="c", subcore_axis_name="s")

  @functools.partial(
      pl.kernel,
      mesh=mesh,
      out_type=jax.ShapeDtypeStruct((_NUM_TOKENS, _HIDDEN), jnp.float32),
      scratch_types=[
          pltpu.VMEM((_NCHUNK, _CHUNK), jnp.int32),
          pltpu.VMEM((_NBUF, _CHUNK, _HIDDEN), jnp.float32),
          pltpu.SemaphoreType.DMA,
          pltpu.SemaphoreType.DMA,
          pltpu.SemaphoreType.DMA,
      ],
  )
  def gather_kernel(idx_hbm, table_hbm, out_hbm, idx_v, rows_v, isem, gsem,
                    ssem):
    wid = lax.axis_index("s") * _NUM_CORES + lax.axis_index("c")
    base = wid * _B_PER_W
    pltpu.async_copy(idx_hbm.at[wid], idx_v, isem).wait()

    gathers = []
    scatters = []

    def start_gather(n):
      gathers.append(
          pltpu.async_copy(table_hbm.at[idx_v.at[n]],
                           rows_v.at[n % _NBUF], gsem))

    for n in range(_LA):
      start_gather(n)
    for c in range(_NCHUNK):
      gathers[c].wait()
      scatters.append(
          pltpu.async_copy(rows_v.at[c % _NBUF],
                           out_hbm.at[pl.ds(base + c * _CHUNK, _CHUNK)],
                           ssem))
      n = c + _LA
      if n < _NCHUNK:
        if n >= _NBUF:
          scatters[n - _NBUF].wait()
        start_gather(n)
    for c in range(_NCHUNK - _NBUF, _NCHUNK):
      scatters[c].wait()

  return gather_kernel


_gather = _make_gather()


def kernel(token_ids, table):
  ids = token_ids.astype(jnp.int32).reshape(_NW, _NCHUNK, _CHUNK)
  out = _gather(ids, table)
  return out.reshape(token_ids.shape[0], token_ids.shape[1], _HIDDEN)
